# baseline (speedup 1.0000x reference)
import functools

import jax
import jax.numpy as jnp
from jax import lax
from jax.experimental import pallas as pl
from jax.experimental.pallas import tpu as pltpu
from jax.experimental.pallas import tpu_sc as plsc

_NUM_WORKERS = 32
_W = 32
_LANES = 16


def _fused_gather_add(xf, idx, emb):
    n, d = xf.shape
    rows_per_worker = n // _NUM_WORKERS
    steps = rows_per_worker // _W
    mesh = plsc.VectorSubcoreMesh(core_axis_name="c", subcore_axis_name="s")

    @functools.partial(
        pl.kernel,
        mesh=mesh,
        out_type=jax.ShapeDtypeStruct((n, d), jnp.float32),
        scratch_types=[
            pltpu.VMEM((_W,), jnp.int32),
            pltpu.VMEM((_W, d), jnp.float32),
            pltpu.VMEM((_W, d), jnp.float32),
            pltpu.SemaphoreType.DMA,
            pltpu.SemaphoreType.DMA,
        ],
    )
    def k(x_hbm, idx_hbm, emb_hbm, out_hbm, idx_v, x_v, rows_v, gsem, xsem):
        wid = lax.axis_index("s") * 2 + lax.axis_index("c")
        base = wid * rows_per_worker

        @pl.loop(0, steps)
        def _(i):
            off = base + i * _W
            pltpu.sync_copy(idx_hbm.at[pl.ds(off, _W)], idx_v)
            gather = pltpu.async_copy(emb_hbm.at[idx_v], rows_v, gsem)
            xcopy = pltpu.async_copy(x_hbm.at[pl.ds(off, _W)], x_v, xsem)
            gather.wait()
            xcopy.wait()

            @pl.loop(0, _W)
            def _(r):
                @pl.loop(0, d, step=_LANES)
                def _(c):
                    x_v[r, pl.ds(c, _LANES)] = (
                        x_v[r, pl.ds(c, _LANES)] + rows_v[r, pl.ds(c, _LANES)]
                    )

            pltpu.sync_copy(x_v, out_hbm.at[pl.ds(off, _W)])

    return k(xf, idx, emb)


def kernel(x, pos_idx, embed):
    b, l, d = x.shape
    xf = x.reshape(b * l, d)
    idx = pos_idx.reshape(-1).astype(jnp.int32)
    emb = embed.reshape(embed.shape[0], d)
    out = _fused_gather_add(xf, idx, emb)
    return out.reshape(b, l, d)


# speedup vs baseline: 1.5926x; 1.5926x over previous
import functools

import jax
import jax.numpy as jnp
from jax import lax
from jax.experimental import pallas as pl
from jax.experimental.pallas import tpu as pltpu
from jax.experimental.pallas import tpu_sc as plsc

_NUM_WORKERS = 32
_W = 16
_LANES = 16


def _fused_gather_add(x, idx, emb):
    nb, nl, d = x.shape
    rows_per_worker = (nb * nl) // _NUM_WORKERS
    workers_per_batch = nl // rows_per_worker
    steps = rows_per_worker // _W
    mesh = plsc.VectorSubcoreMesh(core_axis_name="c", subcore_axis_name="s")

    @functools.partial(
        pl.kernel,
        mesh=mesh,
        out_type=jax.ShapeDtypeStruct((nb, nl, d), jnp.float32),
        scratch_types=[
            pltpu.VMEM((rows_per_worker,), jnp.int32),
            pltpu.VMEM((2, _W, 1, d), jnp.float32),
            pltpu.VMEM((2, _W, d), jnp.float32),
            pltpu.VMEM((2, _W, d), jnp.float32),
            pltpu.SemaphoreType.DMA,
            pltpu.SemaphoreType.DMA,
            pltpu.SemaphoreType.DMA,
            pltpu.SemaphoreType.DMA,
            pltpu.SemaphoreType.DMA,
            pltpu.SemaphoreType.DMA,
        ],
    )
    def k(x_hbm, idx_hbm, emb_hbm, out_hbm, idx_v, rows_v, x_v, o_v,
          gsem0, gsem1, xsem0, xsem1, osem0, osem1):
        gsems = (gsem0, gsem1)
        xsems = (xsem0, xsem1)
        osems = (osem0, osem1)
        wid = lax.axis_index("s") * 2 + lax.axis_index("c")
        bidx = wid // workers_per_batch
        l_base = (wid % workers_per_batch) * rows_per_worker

        pltpu.sync_copy(idx_hbm.at[bidx, pl.ds(l_base, rows_per_worker)], idx_v)

        def issue(loc, b):
            idx_slice = idx_v.at[pl.ds(loc, _W)]
            pltpu.async_copy(emb_hbm.at[idx_slice], rows_v.at[b], gsems[b])
            pltpu.async_copy(x_hbm.at[bidx, pl.ds(l_base + loc, _W)], x_v.at[b],
                             xsems[b])

        for b in range(2):
            issue(b * _W, b)

        @pl.loop(0, steps, step=2)
        def _(i):
            for b in range(2):
                loc = (i + b) * _W
                pltpu.make_async_copy(emb_hbm.at[idx_v.at[pl.ds(loc, _W)]],
                                      rows_v.at[b], gsems[b]).wait()
                pltpu.make_async_copy(x_hbm.at[bidx, pl.ds(l_base + loc, _W)],
                                      x_v.at[b], xsems[b]).wait()
                @pl.when(i >= 2)
                def _():
                    pltpu.make_async_copy(
                        o_v.at[b],
                        out_hbm.at[bidx, pl.ds(l_base + loc - 2 * _W, _W)],
                        osems[b]).wait()

                rb, xb, ob = rows_v.at[b], x_v.at[b], o_v.at[b]

                @pl.loop(0, _W)
                def _(r):
                    for c in range(0, d, _LANES):
                        ob[r, pl.ds(c, _LANES)] = (
                            rb[r, 0, pl.ds(c, _LANES)] + xb[r, pl.ds(c, _LANES)]
                        )

                pltpu.async_copy(o_v.at[b],
                                 out_hbm.at[bidx, pl.ds(l_base + loc, _W)],
                                 osems[b])

                @pl.when(i + 2 < steps)
                def _():
                    issue(loc + 2 * _W, b)

        for b in range(2):
            loc = (steps - 2 + b) * _W
            pltpu.make_async_copy(o_v.at[b],
                                  out_hbm.at[bidx, pl.ds(l_base + loc, _W)],
                                  osems[b]).wait()

    return k(x, idx, emb)


def kernel(x, pos_idx, embed):
    return _fused_gather_add(x, pos_idx.astype(jnp.int32), embed)


# speedup vs baseline: 3.0329x; 1.9044x over previous
import functools

import jax
import jax.numpy as jnp
from jax import lax
from jax.experimental import pallas as pl
from jax.experimental.pallas import tpu as pltpu
from jax.experimental.pallas import tpu_sc as plsc

_NUM_WORKERS = 32
_W = 16
_LANES = 16


def _fused_gather_add(x, idx, emb):
    nb, nl, d = x.shape
    n = nb * nl
    rows_per_worker = n // _NUM_WORKERS
    steps = rows_per_worker // _W
    mesh = plsc.VectorSubcoreMesh(core_axis_name="c", subcore_axis_name="s")

    @functools.partial(
        pl.kernel,
        mesh=mesh,
        out_type=jax.ShapeDtypeStruct((nb, nl, d), jnp.float32),
        scratch_types=[
            pltpu.VMEM((rows_per_worker,), jnp.int32),
            pltpu.VMEM((2, _W, d), jnp.float32),
            pltpu.VMEM((2, _W, d), jnp.float32),
            pltpu.VMEM((2, _W, d), jnp.float32),
            pltpu.SemaphoreType.DMA,
            pltpu.SemaphoreType.DMA,
            pltpu.SemaphoreType.DMA,
            pltpu.SemaphoreType.DMA,
            pltpu.SemaphoreType.DMA,
            pltpu.SemaphoreType.DMA,
        ],
    )
    def k(x_hbm3, idx_hbm, emb_hbm3, out_hbm3, idx_v, rows_v, x_v, o_v,
          gsem0, gsem1, xsem0, xsem1, osem0, osem1):
        x_hbm = x_hbm3.reshape(n, d)
        emb_hbm = emb_hbm3.reshape(emb_hbm3.shape[0], d)
        out_hbm = out_hbm3.reshape(n, d)
        gsems = (gsem0, gsem1)
        xsems = (xsem0, xsem1)
        osems = (osem0, osem1)
        wid = lax.axis_index("s") * 2 + lax.axis_index("c")
        base = wid * rows_per_worker

        pltpu.sync_copy(idx_hbm.at[pl.ds(base, rows_per_worker)], idx_v)

        def issue(loc, b):
            idx_slice = idx_v.at[pl.ds(loc, _W)]
            pltpu.async_copy(emb_hbm.at[idx_slice], rows_v.at[b], gsems[b])
            pltpu.async_copy(x_hbm.at[pl.ds(base + loc, _W)], x_v.at[b],
                             xsems[b])

        for b in range(2):
            issue(b * _W, b)

        @pl.loop(0, steps, step=2)
        def _(i):
            for b in range(2):
                loc = (i + b) * _W
                off = base + loc
                pltpu.make_async_copy(emb_hbm.at[idx_v.at[pl.ds(loc, _W)]],
                                      rows_v.at[b], gsems[b]).wait()
                pltpu.make_async_copy(x_hbm.at[pl.ds(off, _W)], x_v.at[b],
                                      xsems[b]).wait()
                @pl.when(i >= 2)
                def _():
                    pltpu.make_async_copy(
                        o_v.at[b], out_hbm.at[pl.ds(off - 2 * _W, _W)],
                        osems[b]).wait()

                rb, xb, ob = rows_v.at[b], x_v.at[b], o_v.at[b]

                @pl.loop(0, _W)
                def _(r):
                    for c in range(0, d, _LANES):
                        ob[r, pl.ds(c, _LANES)] = (
                            rb[r, pl.ds(c, _LANES)] + xb[r, pl.ds(c, _LANES)]
                        )

                pltpu.async_copy(o_v.at[b], out_hbm.at[pl.ds(off, _W)], osems[b])

                @pl.when(i + 2 < steps)
                def _():
                    issue(loc + 2 * _W, b)

        for b in range(2):
            off = base + (steps - 2 + b) * _W
            pltpu.make_async_copy(o_v.at[b], out_hbm.at[pl.ds(off, _W)],
                                  osems[b]).wait()

    return k(x, idx, emb)


def kernel(x, pos_idx, embed):
    idx = pos_idx.reshape(-1).astype(jnp.int32)
    return _fused_gather_add(x, idx, embed)
